# baseline (speedup 1.0000x reference)
import functools

import jax
import jax.numpy as jnp
from jax import lax
from jax.experimental import pallas as pl
from jax.experimental.pallas import tpu as pltpu
from jax.experimental.pallas import tpu_sc as plsc

NC = 2
NS = 16
NW = NC * NS

DIM = 32
IDX_W = 128
GROUP = 5
NBUF = 4

TAB_BLK = 2048
OUT_BLK = 512


def _table_rows(table_t):
  v = table_t.shape[1]
  grid = pl.cdiv(v, TAB_BLK)

  def body(t_ref, o_ref):
    o_ref[...] = t_ref[...].T

  return pl.pallas_call(
      body,
      out_shape=jax.ShapeDtypeStruct((v, DIM), jnp.float32),
      grid=(grid,),
      in_specs=[pl.BlockSpec((DIM, TAB_BLK), lambda c: (0, c))],
      out_specs=pl.BlockSpec((TAB_BLK, DIM), lambda c: (c, 0)),
  )(table_t)


def _out_transpose(g3):
  l, b, _ = g3.shape

  def body(g_ref, o_ref):
    o_ref[...] = jnp.swapaxes(g_ref[...], 1, 2)

  return pl.pallas_call(
      body,
      out_shape=jax.ShapeDtypeStruct((l, DIM, b), jnp.float32),
      grid=(l, b // OUT_BLK),
      in_specs=[pl.BlockSpec((1, OUT_BLK, DIM), lambda i, c: (i, c, 0))],
      out_specs=pl.BlockSpec((1, DIM, OUT_BLK), lambda i, c: (i, 0, c)),
  )(g3)


def _make_gather(n_total: int):
  rows_per_w = n_total // NW
  idx_rows_w = rows_per_w // IDX_W
  n_trips = idx_rows_w // GROUP
  chunk = GROUP * IDX_W
  assert n_trips % NBUF == 0 and n_trips >= 2 * NBUF

  mesh = plsc.VectorSubcoreMesh(
      core_axis_name="c", subcore_axis_name="s", num_cores=NC,
      num_subcores=NS)

  @functools.partial(
      pl.kernel,
      out_type=jax.ShapeDtypeStruct((n_total, DIM), jnp.float32),
      mesh=mesh,
      scratch_types=[
          pltpu.VMEM((idx_rows_w, IDX_W), jnp.int32),
          [pltpu.VMEM((chunk, DIM), jnp.float32) for _ in range(NBUF)],
          [pltpu.SemaphoreType.DMA for _ in range(NBUF)],
          [pltpu.SemaphoreType.DMA for _ in range(NBUF)],
      ],
      compiler_params=pltpu.CompilerParams(use_tc_tiling_on_sc=False),
  )
  def gather_kernel(table_hbm, idx_hbm, out_hbm, idx_v, bufs, sg, sw):
    wid = lax.axis_index("s") * NC + lax.axis_index("c")
    idx_row_base = wid * idx_rows_w
    out_base = wid * rows_per_w

    pltpu.sync_copy(idx_hbm.at[pl.ds(idx_row_base, idx_rows_w)], idx_v)

    def issue_g(t, s):
      for b in range(GROUP):
        pltpu.async_copy(
            table_hbm.at[idx_v.at[t * GROUP + b]],
            bufs[s].at[pl.ds(b * IDX_W, IDX_W)],
            sg[s],
        )

    def wait_g(s):
      pltpu.make_async_copy(
          table_hbm.at[pl.ds(0, chunk)], bufs[s], sg[s]).wait()

    def issue_w(t, s):
      pltpu.async_copy(
          bufs[s], out_hbm.at[pl.ds(out_base + t * chunk, chunk)], sw[s])

    def wait_w(s):
      pltpu.make_async_copy(
          bufs[s], out_hbm.at[pl.ds(0, chunk)], sw[s]).wait()

    for s in range(NBUF):
      issue_g(s, s)
      if s >= 1:
        wait_g(s - 1)
        issue_w(s - 1, s - 1)

    def outer(o_idx, _):
      o = o_idx * NBUF
      for s in range(NBUF):
        t = o + s
        wait_w(s)
        issue_g(t, s)
        ps = (s - 1) % NBUF
        wait_g(ps)
        issue_w(t - 1, ps)
      return _

    lax.fori_loop(1, n_trips // NBUF, outer, None)

    wait_g(NBUF - 1)
    issue_w(n_trips - 1, NBUF - 1)
    for s in range(NBUF):
      wait_w(s)

  return gather_kernel


def kernel(x, table):
  b, l = x.shape
  n_total = b * l
  rows_tab = _table_rows(table.T)
  idx_t = jnp.swapaxes(x, 0, 1).reshape(n_total // IDX_W, IDX_W)
  gathered = _make_gather(n_total)(rows_tab, idx_t)
  out3 = _out_transpose(gathered.reshape(l, b, DIM))
  return jnp.transpose(out3, (2, 0, 1))


# speedup vs baseline: 2.1946x; 2.1946x over previous
import functools

import jax
import jax.numpy as jnp
from jax import lax
from jax.experimental import pallas as pl
from jax.experimental.pallas import tpu as pltpu
from jax.experimental.pallas import tpu_sc as plsc

NC = 2
NS = 16
NW = NC * NS

DIM = 32
IDX_W = 128
GROUP = 5
NBUF = 4

TAB_BLK = 2048
OUT_BLK = 512


def _table_rows(table_t):
  v = table_t.shape[1]
  grid = pl.cdiv(v, TAB_BLK)

  def body(t_ref, o_ref):
    o_ref[...] = t_ref[...].T

  return pl.pallas_call(
      body,
      out_shape=jax.ShapeDtypeStruct((v, DIM), jnp.float32),
      grid=(grid,),
      in_specs=[pl.BlockSpec((DIM, TAB_BLK), lambda c: (0, c))],
      out_specs=pl.BlockSpec((TAB_BLK, DIM), lambda c: (c, 0)),
  )(table_t)


def _out_transpose(g3):
  l, b, _ = g3.shape

  def body(g_ref, o_ref):
    o_ref[...] = jnp.swapaxes(g_ref[...], 1, 2)

  return pl.pallas_call(
      body,
      out_shape=jax.ShapeDtypeStruct((l, DIM, b), jnp.float32),
      grid=(l, b // OUT_BLK),
      in_specs=[pl.BlockSpec((1, OUT_BLK, DIM), lambda i, c: (i, c, 0))],
      out_specs=pl.BlockSpec((1, DIM, OUT_BLK), lambda i, c: (i, 0, c)),
  )(g3)


def _make_gather(n_total: int):
  rows_per_w = n_total // NW
  idx_rows_w = rows_per_w // IDX_W
  n_trips = idx_rows_w // GROUP
  chunk = GROUP * IDX_W
  assert n_trips % NBUF == 0 and n_trips >= 2 * NBUF

  mesh = plsc.VectorSubcoreMesh(
      core_axis_name="c", subcore_axis_name="s", num_cores=NC,
      num_subcores=NS)

  @functools.partial(
      pl.kernel,
      out_type=jax.ShapeDtypeStruct((n_total, DIM), jnp.float32),
      mesh=mesh,
      scratch_types=[
          pltpu.VMEM((idx_rows_w, IDX_W), jnp.int32),
          [pltpu.VMEM((chunk, DIM), jnp.float32) for _ in range(NBUF)],
          [pltpu.SemaphoreType.DMA for _ in range(NBUF)],
          [pltpu.SemaphoreType.DMA for _ in range(NBUF)],
      ],
      compiler_params=pltpu.CompilerParams(use_tc_tiling_on_sc=False),
  )
  def gather_kernel(table_hbm, idx_hbm, out_hbm, idx_v, bufs, sg, sw):
    wid = lax.axis_index("s") * NC + lax.axis_index("c")
    idx_row_base = wid * idx_rows_w
    out_base = wid * rows_per_w

    pltpu.sync_copy(idx_hbm.at[pl.ds(idx_row_base, idx_rows_w)], idx_v)

    def issue_g(t, s):
      for b in range(GROUP):
        pltpu.async_copy(
            table_hbm.at[idx_v.at[t * GROUP + b]],
            bufs[s].at[pl.ds(b * IDX_W, IDX_W)],
            sg[s],
        )

    def wait_g(s):
      pltpu.make_async_copy(
          table_hbm.at[pl.ds(0, chunk)], bufs[s], sg[s]).wait()

    def issue_w(t, s):
      pltpu.async_copy(
          bufs[s], out_hbm.at[pl.ds(out_base + t * chunk, chunk)], sw[s])

    def wait_w(s):
      pltpu.make_async_copy(
          bufs[s], out_hbm.at[pl.ds(0, chunk)], sw[s]).wait()

    for s in range(NBUF):
      issue_g(s, s)
      if s >= 1:
        wait_g(s - 1)
        issue_w(s - 1, s - 1)

    def outer(o_idx, _):
      o = o_idx * NBUF
      for s in range(NBUF):
        t = o + s
        wait_w(s)
        issue_g(t, s)
        ps = (s - 1) % NBUF
        wait_g(ps)
        issue_w(t - 1, ps)
      return _

    lax.fori_loop(1, n_trips // NBUF, outer, None)

    wait_g(NBUF - 1)
    issue_w(n_trips - 1, NBUF - 1)
    for s in range(NBUF):
      wait_w(s)

  return gather_kernel


def kernel(x, table):
  b, l = x.shape
  n_total = b * l
  idx_t = jnp.swapaxes(x, 0, 1).reshape(n_total // IDX_W, IDX_W)
  gathered = _make_gather(n_total)(table, idx_t)
  return jnp.transpose(gathered.reshape(l, b, DIM), (1, 0, 2))
